# Initial kernel scaffold; baseline (speedup 1.0000x reference)
import jax
import jax.numpy as jnp
from jax.experimental import pallas as pl


def kernel(features, params1, params2, params3, p1, p2, p3, g2m_attr, m2m_attr, m2g_attr, g2m_src, g2m_dst, m2m_src, m2m_dst, m2g_src, m2g_dst):
    raise NotImplementedError("write your pallas kernel here")


# speedup vs baseline: 4.7636x; 4.7636x over previous
import jax
import jax.numpy as jnp
from jax.experimental import pallas as pl

N_GRID_C = 324
N_MESH_C = 81


def _silu(x):
    return x * jax.lax.logistic(x)


def _ln(x, lnp):
    s, b = lnp
    mu = jnp.mean(x, axis=-1, keepdims=True)
    var = jnp.mean(jnp.square(x - mu), axis=-1, keepdims=True)
    return (x - mu) * jax.lax.rsqrt(var + 1e-5) * s[0] + b[0]


def _mlp(p, x):
    layers = p["layers"]
    n = len(layers)
    for li, (Wr, br) in enumerate(layers):
        x = jnp.dot(x, Wr[0, 0], preferred_element_type=jnp.float32) + br[0]
        if li < n - 1:
            x = _silu(x)
    if "ln" in p:
        x = _ln(x, p["ln"])
    return x


def _tail(players, z, pln):
    for li in (1, 2):
        Wr, br = players[li]
        z = jnp.dot(z, Wr[0, 0], preferred_element_type=jnp.float32) + br[0]
        if li < 2:
            z = _silu(z)
    return _ln(z, pln)


def _mp_block(p, h_src, h_dst, e, oh_src, oh_dst, oh_dst_T, zero_dst):
    pe = p["edge"]["layers"]
    W0r, b0r = pe[0]
    W0 = W0r[0, 0]
    z = jnp.dot(
        oh_src,
        jnp.dot(h_src, W0[0:128], preferred_element_type=jnp.float32),
        preferred_element_type=jnp.float32,
    )
    z = z + jnp.dot(e, W0[256:384], preferred_element_type=jnp.float32) + b0r[0]
    if not zero_dst:
        z = z + jnp.dot(
            oh_dst,
            jnp.dot(h_dst, W0[128:256], preferred_element_type=jnp.float32),
            preferred_element_type=jnp.float32,
        )
    z = _silu(z)
    e_new = e + _tail(pe, z, p["edge"]["ln"])

    agg = jnp.dot(oh_dst_T, e_new, preferred_element_type=jnp.float32)

    pn = p["node"]["layers"]
    V0r, c0r = pn[0]
    V0 = V0r[0, 0]
    y = jnp.dot(agg, V0[128:256], preferred_element_type=jnp.float32) + c0r[0]
    if not zero_dst:
        y = y + jnp.dot(h_dst, V0[0:128], preferred_element_type=jnp.float32)
    y = _silu(y)
    y = _tail(pn, y, p["node"]["ln"])
    h_new = y if zero_dst else h_dst + y
    return h_new, e_new


def _onehot(col_ref, n):
    e = col_ref.shape[0]
    ids = jax.lax.broadcasted_iota(jnp.int32, (e, n), 1)
    return (ids == col_ref[:]).astype(jnp.float32)


def _onehot_t(row_ref, n):
    e = row_ref.shape[1]
    ids = jax.lax.broadcasted_iota(jnp.int32, (n, e), 0)
    return (ids == row_ref[:]).astype(jnp.float32)


def _fc_kernel(x_ref, ps_ref, attrs, idx, P, out_ref):
    oh_g2m_src = _onehot(idx["g2m_src_c"], N_GRID_C)
    oh_g2m_dst_t = _onehot_t(idx["g2m_dst_r"], N_MESH_C)
    oh_m2m_src = _onehot(idx["m2m_src_c"], N_MESH_C)
    oh_m2m_dst = _onehot(idx["m2m_dst_c"], N_MESH_C)
    oh_m2m_dst_t = _onehot_t(idx["m2m_dst_r"], N_MESH_C)
    oh_m2g_src = _onehot(idx["m2g_src_c"], N_MESH_C)
    oh_m2g_dst = _onehot(idx["m2g_dst_c"], N_GRID_C)
    oh_m2g_dst_t = _onehot_t(idx["m2g_dst_r"], N_GRID_C)

    x = x_ref[0]
    h_g = _mlp(P["enc_node"], x)
    e = _mlp(P["enc_edge"], attrs["g2m"][:])
    h_m, e = _mp_block(
        P["enc_blk"], h_g, None, e, oh_g2m_src, None, oh_g2m_dst_t, zero_dst=True
    )
    em = _mlp(P["m2m_edge"], attrs["m2m"][:])
    for bi in range(3):
        h_m, em = _mp_block(
            P["proc"][bi], h_m, h_m, em,
            oh_m2m_src, oh_m2m_dst, oh_m2m_dst_t, zero_dst=False,
        )
    ed = _mlp(P["dec_edge"], attrs["m2g"][:])
    h_g, ed = _mp_block(
        P["dec_blk"], h_m, h_g, ed,
        oh_m2g_src, oh_m2g_dst, oh_m2g_dst_t, zero_dst=False,
    )
    out = x + _mlp(P["dec_out"], h_g)
    w = ps_ref[0]
    contrib = out * w
    i = pl.program_id(0)

    @pl.when(i == 0)
    def _():
        out_ref[:] = contrib

    @pl.when(i > 0)
    def _():
        out_ref[:] = out_ref[:] + contrib


def _stack3(a, b, c):
    return jnp.stack([a, b, c]).reshape((3, 1) + a.shape)


def kernel(features, params1, params2, params3, p1, p2, p3, g2m_attr, m2m_attr,
           m2g_attr, g2m_src, g2m_dst, m2m_src, m2m_dst, m2g_src, m2g_dst):
    f3 = features[0]
    ps = jnp.stack([p1, p2, p3]).astype(jnp.float32).reshape(3, 1, 1)
    P = jax.tree.map(_stack3, params1, params2, params3)
    attrs = {"g2m": g2m_attr, "m2m": m2m_attr, "m2g": m2g_attr}
    idx = {
        "g2m_src_c": g2m_src.reshape(-1, 1),
        "g2m_dst_r": g2m_dst.reshape(1, -1),
        "m2m_src_c": m2m_src.reshape(-1, 1),
        "m2m_dst_c": m2m_dst.reshape(-1, 1),
        "m2m_dst_r": m2m_dst.reshape(1, -1),
        "m2g_src_c": m2g_src.reshape(-1, 1),
        "m2g_dst_c": m2g_dst.reshape(-1, 1),
        "m2g_dst_r": m2g_dst.reshape(1, -1),
    }

    def _const_spec(a):
        nd = a.ndim
        return pl.BlockSpec(a.shape, lambda i, _nd=nd: (0,) * _nd)

    def _model_spec(a):
        nd = a.ndim
        return pl.BlockSpec(
            (1,) + a.shape[1:], lambda i, _nd=nd: (i,) + (0,) * (_nd - 1)
        )

    spec_x = pl.BlockSpec((1, N_GRID_C, 42), lambda i: (i, 0, 0))
    spec_ps = pl.BlockSpec((1, 1, 1), lambda i: (i, 0, 0))
    attrs_specs = jax.tree.map(_const_spec, attrs)
    idx_specs = jax.tree.map(_const_spec, idx)
    p_specs = jax.tree.map(_model_spec, P)

    out = pl.pallas_call(
        _fc_kernel,
        grid=(3,),
        in_specs=[spec_x, spec_ps, attrs_specs, idx_specs, p_specs],
        out_specs=pl.BlockSpec((N_GRID_C, 42), lambda i: (0, 0)),
        out_shape=jax.ShapeDtypeStruct((N_GRID_C, 42), jnp.float32),
    )(f3, ps, attrs, idx, P)
    return out[None]


# speedup vs baseline: 11.1448x; 2.3396x over previous
import jax
import jax.numpy as jnp
from jax.experimental import pallas as pl

N_GRID_C = 324
N_MESH_C = 81


def _silu(x):
    return x * jax.lax.logistic(x)


def _ln(x, lnp):
    s, b = lnp
    mu = jnp.mean(x, axis=-1, keepdims=True)
    var = jnp.mean(jnp.square(x - mu), axis=-1, keepdims=True)
    return (x - mu) * jax.lax.rsqrt(var + 1e-5) * s[:] + b[:]


def _mlp(p, x):
    layers = p["layers"]
    n = len(layers)
    for li, (Wr, br) in enumerate(layers):
        x = jnp.dot(x, Wr[:], preferred_element_type=jnp.float32) + br[:]
        if li < n - 1:
            x = _silu(x)
    if "ln" in p:
        x = _ln(x, p["ln"])
    return x


def _tail(players, z, pln):
    for li in (1, 2):
        Wr, br = players[li]
        z = jnp.dot(z, Wr[:], preferred_element_type=jnp.float32) + br[:]
        if li < 2:
            z = _silu(z)
    return _ln(z, pln)


def _mp_block(p, h_src, h_dst, e, oh_src, oh_dst, oh_dst_T, zero_dst):
    pe = p["edge"]["layers"]
    W0r, b0r = pe[0]
    z = jnp.dot(
        oh_src,
        jnp.dot(h_src, W0r[0:128], preferred_element_type=jnp.float32),
        preferred_element_type=jnp.float32,
    )
    z = z + jnp.dot(e, W0r[256:384], preferred_element_type=jnp.float32) + b0r[:]
    if not zero_dst:
        z = z + jnp.dot(
            oh_dst,
            jnp.dot(h_dst, W0r[128:256], preferred_element_type=jnp.float32),
            preferred_element_type=jnp.float32,
        )
    z = _silu(z)
    e_new = e + _tail(pe, z, p["edge"]["ln"])

    agg = jnp.dot(oh_dst_T, e_new, preferred_element_type=jnp.float32)

    pn = p["node"]["layers"]
    V0r, c0r = pn[0]
    y = jnp.dot(agg, V0r[128:256], preferred_element_type=jnp.float32) + c0r[:]
    if not zero_dst:
        y = y + jnp.dot(h_dst, V0r[0:128], preferred_element_type=jnp.float32)
    y = _silu(y)
    y = _tail(pn, y, p["node"]["ln"])
    h_new = y if zero_dst else h_dst + y
    return h_new, e_new


def _onehot(col_ref, n):
    e = col_ref.shape[0]
    ids = jax.lax.broadcasted_iota(jnp.int32, (e, n), 1)
    return (ids == col_ref[:]).astype(jnp.float32)


def _onehot_t(row_ref, n):
    e = row_ref.shape[1]
    ids = jax.lax.broadcasted_iota(jnp.int32, (n, e), 0)
    return (ids == row_ref[:]).astype(jnp.float32)


def _forecaster(P, x, attrs, oh):
    h_g = _mlp(P["enc_node"], x)
    e = _mlp(P["enc_edge"], attrs["g2m"][:])
    h_m, e = _mp_block(
        P["enc_blk"], h_g, None, e, oh["g2m_src"], None, oh["g2m_dst_t"],
        zero_dst=True,
    )
    em = _mlp(P["m2m_edge"], attrs["m2m"][:])
    for bi in range(3):
        h_m, em = _mp_block(
            P["proc"][bi], h_m, h_m, em,
            oh["m2m_src"], oh["m2m_dst"], oh["m2m_dst_t"], zero_dst=False,
        )
    ed = _mlp(P["dec_edge"], attrs["m2g"][:])
    h_g, ed = _mp_block(
        P["dec_blk"], h_m, h_g, ed,
        oh["m2g_src"], oh["m2g_dst"], oh["m2g_dst_t"], zero_dst=False,
    )
    return x + _mlp(P["dec_out"], h_g)


def _fc_kernel(x_ref, ps_ref, attrs, idx, P1, P2, P3, out_ref):
    oh = {
        "g2m_src": _onehot(idx["g2m_src_c"], N_GRID_C),
        "g2m_dst_t": _onehot_t(idx["g2m_dst_r"], N_MESH_C),
        "m2m_src": _onehot(idx["m2m_src_c"], N_MESH_C),
        "m2m_dst": _onehot(idx["m2m_dst_c"], N_MESH_C),
        "m2m_dst_t": _onehot_t(idx["m2m_dst_r"], N_MESH_C),
        "m2g_src": _onehot(idx["m2g_src_c"], N_MESH_C),
        "m2g_dst": _onehot(idx["m2g_dst_c"], N_GRID_C),
        "m2g_dst_t": _onehot_t(idx["m2g_dst_r"], N_GRID_C),
    }
    acc = None
    for mi, P in enumerate((P1, P2, P3)):
        x = x_ref[0, mi]
        o = _forecaster(P, x, attrs, oh)
        w = ps_ref[mi : mi + 1, :]
        acc = o * w if acc is None else acc + o * w
    out_ref[:] = acc


def kernel(features, params1, params2, params3, p1, p2, p3, g2m_attr, m2m_attr,
           m2g_attr, g2m_src, g2m_dst, m2m_src, m2m_dst, m2g_src, m2g_dst):
    ps = jnp.stack([p1, p2, p3]).astype(jnp.float32).reshape(3, 1)
    attrs = {"g2m": g2m_attr, "m2m": m2m_attr, "m2g": m2g_attr}
    idx = {
        "g2m_src_c": g2m_src.reshape(-1, 1),
        "g2m_dst_r": g2m_dst.reshape(1, -1),
        "m2m_src_c": m2m_src.reshape(-1, 1),
        "m2m_dst_c": m2m_dst.reshape(-1, 1),
        "m2m_dst_r": m2m_dst.reshape(1, -1),
        "m2g_src_c": m2g_src.reshape(-1, 1),
        "m2g_dst_c": m2g_dst.reshape(-1, 1),
        "m2g_dst_r": m2g_dst.reshape(1, -1),
    }
    out = pl.pallas_call(
        _fc_kernel,
        out_shape=jax.ShapeDtypeStruct((N_GRID_C, 42), jnp.float32),
    )(features, ps, attrs, idx, params1, params2, params3)
    return out[None]
